# Initial kernel scaffold; baseline (speedup 1.0000x reference)
import jax
import jax.numpy as jnp
from jax.experimental import pallas as pl


def kernel(h, e_w, snorm_n, snorm_e, edge_index, Wh, bh, We, be, Wself1, Wfunc1, Wattn1, Wself2, Wfunc2, Wattn2, W1, b1):
    raise NotImplementedError("write your pallas kernel here")


# speedup vs baseline: 8.4635x; 8.4635x over previous
import functools

import jax
import jax.numpy as jnp
from jax import lax
from jax.experimental import pallas as pl
from jax.experimental.pallas import tpu as pltpu
from jax.experimental.pallas import tpu_sc as plsc

N = 10000
E = 320000
D_IN = 128
DH = 256
DHH = 128
D_OUT = 128

NC = 2
NS = 16
L = 16

EPT = E // NS
B = 80
NBLK = EPT // B
NPAD = 10240
WPT = NPAD // NS
SROWS = NPAD // 128

BN = 1000


def _f32(x):
    return jnp.asarray(x, jnp.float32)


def _mm_t(x, w):
    return lax.dot_general(x, w, (((1,), (1,)), ((), ())),
                           preferred_element_type=jnp.float32)


def _stage0_body(h_ref, Wh_ref, bh_ref, Ws_ref, Wf_ref, wa_ref,
                 h0_ref, hs_ref, z_ref, aa_ref):
    h0 = _mm_t(h_ref[...], Wh_ref[...]) + bh_ref[...][None, :]
    h0_ref[...] = h0
    hs_ref[...] = _mm_t(h0, Ws_ref[...])
    z = _mm_t(h0, Wf_ref[...])
    z_ref[...] = z
    aa_ref[...] = jnp.dot(z, wa_ref[...], preferred_element_type=jnp.float32)


def _combine(x, hs, m0, m1, s):
    sgood = s > 0.0
    inv = 1.0 / jnp.where(sgood, s, 1.0)
    msg = jnp.concatenate([m0 * inv, m1 * inv], axis=1)
    upd = jnp.where(sgood, hs + msg, x)
    return x + jnp.maximum(upd, 0.0)


def _stage1_body(x_ref, hs1_ref, m0_ref, m1_ref, s_ref, Ws_ref, Wf_ref, wa_ref,
                 h1_ref, hs_ref, z_ref, aa_ref):
    h1 = _combine(x_ref[...], hs1_ref[...], m0_ref[...], m1_ref[...], s_ref[...])
    h1_ref[...] = h1
    hs_ref[...] = _mm_t(h1, Ws_ref[...])
    z = _mm_t(h1, Wf_ref[...])
    z_ref[...] = z
    aa_ref[...] = jnp.dot(z, wa_ref[...], preferred_element_type=jnp.float32)


def _stage2_body(x_ref, hs2_ref, m0_ref, m1_ref, s_ref, W1_ref, b1_ref, y_ref):
    h2 = _combine(x_ref[...], hs2_ref[...], m0_ref[...], m1_ref[...], s_ref[...])
    y_ref[...] = _mm_t(h2, W1_ref[...]) + b1_ref[...][None, :]


def _row_spec(width):
    return pl.BlockSpec((BN, width), lambda i: (i, 0))


def _w_spec(shape):
    return pl.BlockSpec(shape, lambda i: (0,) * len(shape))


def _stage0(h, Wh, bh, Wself, Wfunc, wa):
    return pl.pallas_call(
        _stage0_body,
        grid=(N // BN,),
        in_specs=[_row_spec(D_IN), _w_spec(Wh.shape), _w_spec(bh.shape),
                  _w_spec(Wself.shape), _w_spec(Wfunc.shape), _w_spec(wa.shape)],
        out_specs=[_row_spec(DH), _row_spec(DH), _row_spec(DH), _row_spec(2)],
        out_shape=[jax.ShapeDtypeStruct((N, DH), jnp.float32),
                   jax.ShapeDtypeStruct((N, DH), jnp.float32),
                   jax.ShapeDtypeStruct((N, DH), jnp.float32),
                   jax.ShapeDtypeStruct((N, 2), jnp.float32)],
    )(h, Wh, bh, Wself, Wfunc, wa)


def _stage1(x, hs1, msg0, msg1, s, Wself, Wfunc, wa):
    return pl.pallas_call(
        _stage1_body,
        grid=(N // BN,),
        in_specs=[_row_spec(DH), _row_spec(DH), _row_spec(DHH), _row_spec(DHH),
                  _row_spec(1),
                  _w_spec(Wself.shape), _w_spec(Wfunc.shape), _w_spec(wa.shape)],
        out_specs=[_row_spec(DH), _row_spec(DH), _row_spec(DH), _row_spec(2)],
        out_shape=[jax.ShapeDtypeStruct((N, DH), jnp.float32),
                   jax.ShapeDtypeStruct((N, DH), jnp.float32),
                   jax.ShapeDtypeStruct((N, DH), jnp.float32),
                   jax.ShapeDtypeStruct((N, 2), jnp.float32)],
    )(x, hs1, msg0, msg1, s, Wself, Wfunc, wa)


def _stage2(x, hs2, msg0, msg1, s, W1, b1):
    return pl.pallas_call(
        _stage2_body,
        grid=(N // BN,),
        in_specs=[_row_spec(DH), _row_spec(DH), _row_spec(DHH), _row_spec(DHH),
                  _row_spec(1),
                  _w_spec(W1.shape), _w_spec(b1.shape)],
        out_specs=_row_spec(D_OUT),
        out_shape=jax.ShapeDtypeStruct((N, D_OUT), jnp.float32),
    )(x, hs2, msg0, msg1, s, W1, b1)


def _sc_body(zcat, asv, adv, src, dst, ew, c1v_h, c0v_h,
             msg0_out, msg1_out, s_out,
             as_l, ad_l, s_l, src_b, dst_b, ew_b, p_b, rows,
             idn, cvec, msg_acc, s_sh, sem):
    c = lax.axis_index("c")
    t = lax.axis_index("s")

    zeros16 = jnp.zeros((L,), jnp.float32)

    def _rows_row(i, _):
        for j in range(DHH // L):
            rows[i, pl.ds(j * L, L)] = zeros16
        return 0
    lax.fori_loop(0, B, _rows_row, 0)

    def _sl_row(i, _):
        for j in range(DHH // L):
            s_l[i, pl.ds(j * L, L)] = zeros16
        return 0
    lax.fori_loop(0, SROWS, _sl_row, 0)

    base_iota = lax.broadcasted_iota(jnp.int32, (L,), 0)
    for g in range(SROWS // L):
        idn[pl.ds(g * L, L)] = base_iota + (g * L)

    @pl.when(t < NS - 1)
    def _():
        for k in range(WPT // 80):
            pltpu.sync_copy(rows, msg_acc.at[pl.ds(t * WPT + k * 80, 80)])

    @pl.when(t == NS - 1)
    def _():
        for k in range((N - (NS - 1) * WPT) // 80):
            pltpu.sync_copy(rows, msg_acc.at[pl.ds((NS - 1) * WPT + k * 80, 80)])

    @pl.when(t < 10)
    def _():
        pltpu.sync_copy(rows.at[pl.ds(0, 8)], s_sh.at[pl.ds(t * 8, 8)])

    pltpu.sync_copy(asv, as_l)
    pltpu.sync_copy(adv, ad_l)
    pltpu.sync_copy(c1v_h, cvec.at[0])
    pltpu.sync_copy(c0v_h, cvec.at[1])
    c1vec = cvec[0, :]
    c0vec = cvec[1, :]

    plsc.subcore_barrier()

    eoff0 = t * EPT
    zbase = c * N

    def _block(b, _):
        eoff = eoff0 + b * B
        pltpu.sync_copy(src.at[pl.ds(eoff, B)], src_b)
        pltpu.sync_copy(dst.at[pl.ds(eoff, B)], dst_b)
        pltpu.sync_copy(ew.at[pl.ds(eoff, B)], ew_b)

        for g in range(B // L):
            sl = pl.ds(g * L, L)
            srcv = src_b[sl]
            dstv = dst_b[sl]
            a_s = plsc.load_gather(as_l, [srcv])
            a_d = plsc.load_gather(ad_l, [dstv])
            sc_ = a_s + a_d + ew_b[sl] * c1vec + c0vec
            lr = jnp.where(sc_ >= 0.0, sc_, sc_ * 0.01)
            p = jnp.exp(lr)
            p_b[sl] = p
            plsc.addupdate_scatter(s_l, [dstv >> 7, dstv & 127], p)
            src_b[sl] = srcv + zbase

        pltpu.async_copy(zcat.at[src_b], rows, sem).wait()

        for g in range(B // L):
            pv = p_b[pl.ds(g * L, L)]
            for i in range(L):
                spl = jnp.broadcast_to(pv[i], (L,))
                r = g * L + i
                for j in range(DHH // L):
                    rows[r, pl.ds(j * L, L)] = rows[r, pl.ds(j * L, L)] * spl

        pltpu.sync_copy(rows, msg_acc.at[dst_b], add=True)
        return 0

    lax.fori_loop(0, NBLK, _block, 0)

    plsc.subcore_barrier()
    pltpu.sync_copy(s_l, s_sh.at[idn], add=True)
    plsc.subcore_barrier()

    tail = N - (NS - 1) * WPT
    for half, mref in ((0, msg0_out), (1, msg1_out)):
        @pl.when((c == half) & (t < NS - 1))
        def _(mref=mref):
            pltpu.sync_copy(msg_acc.at[pl.ds(t * WPT, WPT)],
                            mref.at[pl.ds(t * WPT, WPT)])

        @pl.when((c == half) & (t == NS - 1))
        def _(mref=mref):
            pltpu.sync_copy(msg_acc.at[pl.ds((NS - 1) * WPT, tail)],
                            mref.at[pl.ds((NS - 1) * WPT, tail)])

    @pl.when((c == 0) & (t < 10))
    def _():
        pltpu.sync_copy(s_sh.at[pl.ds(t * 8, 8)], s_out.at[pl.ds(t * 8, 8)])


_sc_call = functools.partial(
    pl.kernel,
    out_type=(jax.ShapeDtypeStruct((NPAD, DHH), jnp.float32),
              jax.ShapeDtypeStruct((NPAD, DHH), jnp.float32),
              jax.ShapeDtypeStruct((SROWS, 128), jnp.float32)),
    mesh=plsc.VectorSubcoreMesh(core_axis_name="c", subcore_axis_name="s",
                                num_cores=NC, num_subcores=NS),
    compiler_params=pltpu.CompilerParams(needs_layout_passes=False),
    scratch_types=[
        pltpu.VMEM((N,), jnp.float32),
        pltpu.VMEM((N,), jnp.float32),
        pltpu.VMEM((SROWS, 128), jnp.float32),
        pltpu.VMEM((B,), jnp.int32),
        pltpu.VMEM((B,), jnp.int32),
        pltpu.VMEM((B,), jnp.float32),
        pltpu.VMEM((B,), jnp.float32),
        pltpu.VMEM((B, DHH), jnp.float32),
        pltpu.VMEM((SROWS,), jnp.int32),
        pltpu.VMEM((2, L), jnp.float32),
        pltpu.VMEM_SHARED((N, DHH), jnp.float32),
        pltpu.VMEM_SHARED((SROWS, 128), jnp.float32),
        pltpu.SemaphoreType.DMA,
    ],
)(_sc_body)


def _sc_aggregate(z, asv, adv, src, dst, ew, c1v, c0v):
    zcat = jnp.concatenate([z[:, :DHH], z[:, DHH:]], axis=0)
    msg0, msg1, s2d = _sc_call(zcat, asv, adv, src, dst, ew, c1v, c0v)
    s = s2d.reshape(NPAD)[:N].reshape(N, 1)
    return msg0, msg1, s


def kernel(h, e_w, snorm_n, snorm_e, edge_index, Wh, bh, We, be,
           Wself1, Wfunc1, Wattn1, Wself2, Wfunc2, Wattn2, W1, b1):
    src = edge_index[0]
    dst = edge_index[1]
    ew = e_w[:, 0]

    wa1 = jnp.stack([Wattn1[0, :DH], Wattn1[0, DH:2 * DH]], axis=1)
    wa2 = jnp.stack([Wattn2[0, :DH], Wattn2[0, DH:2 * DH]], axis=1)
    c1_1 = Wattn1[0, 2 * DH:] @ We[:, 0]
    c0_1 = Wattn1[0, 2 * DH:] @ be
    c1_2 = Wattn2[0, 2 * DH:] @ We[:, 0]
    c0_2 = Wattn2[0, 2 * DH:] @ be
    c1v1 = jnp.full((L,), c1_1, jnp.float32)
    c0v1 = jnp.full((L,), c0_1, jnp.float32)
    c1v2 = jnp.full((L,), c1_2, jnp.float32)
    c0v2 = jnp.full((L,), c0_2, jnp.float32)

    h0, hs1, z1, aa1 = _stage0(h, Wh, bh, Wself1, Wfunc1, wa1)
    m1a, m1b, s1 = _sc_aggregate(z1, aa1[:, 0], aa1[:, 1], src, dst, ew,
                                 c1v1, c0v1)
    h1, hs2, z2, aa2 = _stage1(h0, hs1, m1a, m1b, s1, Wself2, Wfunc2, wa2)
    m2a, m2b, s2 = _sc_aggregate(z2, aa2[:, 0], aa2[:, 1], src, dst, ew,
                                 c1v2, c0v2)
    y = _stage2(h1, hs2, m2a, m2b, s2, W1, b1)
    return y


# speedup vs baseline: 15.5288x; 1.8348x over previous
import functools

import jax
import jax.numpy as jnp
from jax import lax
from jax.experimental import pallas as pl
from jax.experimental.pallas import tpu as pltpu
from jax.experimental.pallas import tpu_sc as plsc

N = 10000
E = 320000
D_IN = 128
DH = 256
DHH = 128
D_OUT = 128

NC = 2
NS = 16
L = 16

EVALID = E // NS
B = 64
NBLK = 314
EPT = NBLK * B
E_PAD = EPT * NS
NPAD = 10240
WPT = NPAD // NS
SROWS = NPAD // 128

BN = 1000


def _f32(x):
    return jnp.asarray(x, jnp.float32)


def _mm_t(x, w):
    return lax.dot_general(x, w, (((1,), (1,)), ((), ())),
                           preferred_element_type=jnp.float32)


def _stage0_body(h_ref, Wh_ref, bh_ref, Ws_ref, Wf_ref, wa_ref,
                 h0_ref, hs_ref, z_ref, aa_ref):
    h0 = _mm_t(h_ref[...], Wh_ref[...]) + bh_ref[...][None, :]
    h0_ref[...] = h0
    hs_ref[...] = _mm_t(h0, Ws_ref[...])
    z = _mm_t(h0, Wf_ref[...])
    z_ref[...] = z
    aa_ref[...] = jnp.dot(z, wa_ref[...], preferred_element_type=jnp.float32)


def _combine(x, hs, m0, m1, s):
    sgood = s > 0.0
    inv = 1.0 / jnp.where(sgood, s, 1.0)
    msg = jnp.concatenate([m0 * inv, m1 * inv], axis=1)
    upd = jnp.where(sgood, hs + msg, x)
    return x + jnp.maximum(upd, 0.0)


def _stage1_body(x_ref, hs1_ref, m0_ref, m1_ref, s_ref, Ws_ref, Wf_ref, wa_ref,
                 h1_ref, hs_ref, z_ref, aa_ref):
    h1 = _combine(x_ref[...], hs1_ref[...], m0_ref[...], m1_ref[...], s_ref[...])
    h1_ref[...] = h1
    hs_ref[...] = _mm_t(h1, Ws_ref[...])
    z = _mm_t(h1, Wf_ref[...])
    z_ref[...] = z
    aa_ref[...] = jnp.dot(z, wa_ref[...], preferred_element_type=jnp.float32)


def _stage2_body(x_ref, hs2_ref, m0_ref, m1_ref, s_ref, W1_ref, b1_ref, y_ref):
    h2 = _combine(x_ref[...], hs2_ref[...], m0_ref[...], m1_ref[...], s_ref[...])
    y_ref[...] = _mm_t(h2, W1_ref[...]) + b1_ref[...][None, :]


def _row_spec(width):
    return pl.BlockSpec((BN, width), lambda i: (i, 0))


def _w_spec(shape):
    return pl.BlockSpec(shape, lambda i: (0,) * len(shape))


def _stage0(h, Wh, bh, Wself, Wfunc, wa):
    return pl.pallas_call(
        _stage0_body,
        grid=(N // BN,),
        in_specs=[_row_spec(D_IN), _w_spec(Wh.shape), _w_spec(bh.shape),
                  _w_spec(Wself.shape), _w_spec(Wfunc.shape), _w_spec(wa.shape)],
        out_specs=[_row_spec(DH), _row_spec(DH), _row_spec(DH), _row_spec(2)],
        out_shape=[jax.ShapeDtypeStruct((N, DH), jnp.float32),
                   jax.ShapeDtypeStruct((N, DH), jnp.float32),
                   jax.ShapeDtypeStruct((N, DH), jnp.float32),
                   jax.ShapeDtypeStruct((N, 2), jnp.float32)],
    )(h, Wh, bh, Wself, Wfunc, wa)


def _stage1(x, hs1, msg0, msg1, s, Wself, Wfunc, wa):
    return pl.pallas_call(
        _stage1_body,
        grid=(N // BN,),
        in_specs=[_row_spec(DH), _row_spec(DH), _row_spec(DHH), _row_spec(DHH),
                  _row_spec(1),
                  _w_spec(Wself.shape), _w_spec(Wfunc.shape), _w_spec(wa.shape)],
        out_specs=[_row_spec(DH), _row_spec(DH), _row_spec(DH), _row_spec(2)],
        out_shape=[jax.ShapeDtypeStruct((N, DH), jnp.float32),
                   jax.ShapeDtypeStruct((N, DH), jnp.float32),
                   jax.ShapeDtypeStruct((N, DH), jnp.float32),
                   jax.ShapeDtypeStruct((N, 2), jnp.float32)],
    )(x, hs1, msg0, msg1, s, Wself, Wfunc, wa)


def _stage2(x, hs2, msg0, msg1, s, W1, b1):
    return pl.pallas_call(
        _stage2_body,
        grid=(N // BN,),
        in_specs=[_row_spec(DH), _row_spec(DH), _row_spec(DHH), _row_spec(DHH),
                  _row_spec(1),
                  _w_spec(W1.shape), _w_spec(b1.shape)],
        out_specs=_row_spec(D_OUT),
        out_shape=jax.ShapeDtypeStruct((N, D_OUT), jnp.float32),
    )(x, hs2, msg0, msg1, s, W1, b1)


def _sc_body(zcat, edata, asv, adv, c1v_h, c0v_h,
             msg0_out, msg1_out, s_out,
             as_l, ad_l, s_l,
             ed0, ed1, p0, p1, si0, si1, rows0, rows1,
             idn, cvec,
             msg_acc, s_sh,
             esem0, esem1, gsem0, gsem1, ssem0, ssem1):
    c = lax.axis_index("c")
    t = lax.axis_index("s")

    eds = (ed0, ed1)
    pbs = (p0, p1)
    sis = (si0, si1)
    rws = (rows0, rows1)
    esems = (esem0, esem1)
    gsems = (gsem0, gsem1)
    ssems = (ssem0, ssem1)

    zeros16 = jnp.zeros((L,), jnp.float32)

    def _rows_row(i, _):
        for j in range(DHH // L):
            rows0[i, pl.ds(j * L, L)] = zeros16
        return 0
    lax.fori_loop(0, B, _rows_row, 0)

    def _sl_row(i, _):
        for j in range(DHH // L):
            s_l[i, pl.ds(j * L, L)] = zeros16
        return 0
    lax.fori_loop(0, SROWS, _sl_row, 0)

    base_iota = lax.broadcasted_iota(jnp.int32, (L,), 0)
    for g in range(SROWS // L):
        idn[pl.ds(g * L, L)] = base_iota + (g * L)

    @pl.when(t < NS - 1)
    def _():
        for k in range(WPT // B):
            pltpu.sync_copy(rows0, msg_acc.at[pl.ds(t * WPT + k * B, B)])

    @pl.when(t == NS - 1)
    def _():
        tail0 = N - (NS - 1) * WPT
        for k in range(tail0 // B):
            pltpu.sync_copy(rows0, msg_acc.at[pl.ds((NS - 1) * WPT + k * B, B)])
        rem = tail0 % B
        if rem:
            pltpu.sync_copy(rows0.at[pl.ds(0, rem)],
                            msg_acc.at[pl.ds((NS - 1) * WPT + (tail0 // B) * B,
                                             rem)])

    @pl.when(t < 10)
    def _():
        pltpu.sync_copy(rows0.at[pl.ds(0, 8)], s_sh.at[pl.ds(t * 8, 8)])

    pltpu.sync_copy(asv, as_l)
    pltpu.sync_copy(adv, ad_l)
    pltpu.sync_copy(c1v_h, cvec.at[0])
    pltpu.sync_copy(c0v_h, cvec.at[1])
    c1vec = cvec[0, :]
    c0vec = cvec[1, :]

    plsc.subcore_barrier()

    eoff0 = t * EPT
    zbase = c * N

    def issue_edges(b, k):
        pltpu.async_copy(edata.at[pl.ds((t * NBLK + b) * 3 * B, 3 * B)],
                         eds[k], esems[k])

    def wait_edges(b, k):
        pltpu.make_async_copy(edata.at[pl.ds((t * NBLK + b) * 3 * B, 3 * B)],
                              eds[k], esems[k]).wait()

    def scalar_phase(b, k):
        ed = eds[k]
        for g in range(B // L):
            sl = pl.ds(g * L, L)
            srcv = ed[pl.ds(g * L, L)]
            dstv = ed[pl.ds(B + g * L, L)]
            eww = plsc.bitcast(ed[pl.ds(2 * B + g * L, L)], jnp.int32)
            a_s = plsc.load_gather(as_l, [srcv])
            a_d = plsc.load_gather(ad_l, [dstv])
            sc_ = a_s + a_d + eww * c1vec + c0vec
            lr = jnp.where(sc_ >= 0.0, sc_, sc_ * 0.01)
            p = jnp.exp(lr)
            lid = b * B + g * L + base_iota
            p = jnp.where(lid < EVALID, p, 0.0)
            pbs[k][sl] = p
            plsc.addupdate_scatter(s_l, [dstv >> 7, dstv & 127], p)
            sis[k][sl] = dstv
            ed[pl.ds(g * L, L)] = srcv + zbase
        pltpu.async_copy(zcat.at[ed.at[pl.ds(0, B)]], rws[k], gsems[k])

    def wait_gather(k):
        pltpu.make_async_copy(zcat.at[eds[k].at[pl.ds(0, B)]], rws[k],
                              gsems[k]).wait()

    def scale_and_scatter(k):
        rw = rws[k]
        for g in range(B // L):
            pv = pbs[k][pl.ds(g * L, L)]
            for i in range(L):
                spl = jnp.broadcast_to(pv[i], (L,))
                r = g * L + i
                for j in range(DHH // L):
                    rw[r, pl.ds(j * L, L)] = rw[r, pl.ds(j * L, L)] * spl
        pltpu.async_copy(rw, msg_acc.at[sis[k]], ssems[k], add=True)

    def wait_scatter(k):
        pltpu.make_async_copy(rws[k], msg_acc.at[sis[k]], ssems[k]).wait()

    def stage(j, A, Bn):
        wait_gather(A)

        @pl.when(j >= 1)
        def _():
            wait_scatter(Bn)

        @pl.when(j + 1 < NBLK)
        def _():
            wait_edges(j + 1, Bn)
            scalar_phase(j + 1, Bn)

        scale_and_scatter(A)

        @pl.when(j + 2 < NBLK)
        def _():
            issue_edges(j + 2, A)

    issue_edges(0, 0)
    issue_edges(1, 1)
    wait_edges(0, 0)
    scalar_phase(0, 0)

    def _pair(i, _):
        stage(2 * i, 0, 1)
        stage(2 * i + 1, 1, 0)
        return 0
    lax.fori_loop(0, NBLK // 2, _pair, 0)

    wait_scatter(1)

    plsc.subcore_barrier()
    pltpu.sync_copy(s_l, s_sh.at[idn], add=True)
    plsc.subcore_barrier()

    tail = N - (NS - 1) * WPT
    for half, mref in ((0, msg0_out), (1, msg1_out)):
        @pl.when((c == half) & (t < NS - 1))
        def _(mref=mref):
            pltpu.sync_copy(msg_acc.at[pl.ds(t * WPT, WPT)],
                            mref.at[pl.ds(t * WPT, WPT)])

        @pl.when((c == half) & (t == NS - 1))
        def _(mref=mref):
            pltpu.sync_copy(msg_acc.at[pl.ds((NS - 1) * WPT, tail)],
                            mref.at[pl.ds((NS - 1) * WPT, tail)])

    @pl.when((c == 0) & (t < 10))
    def _():
        pltpu.sync_copy(s_sh.at[pl.ds(t * 8, 8)], s_out.at[pl.ds(t * 8, 8)])


_sc_call = functools.partial(
    pl.kernel,
    out_type=(jax.ShapeDtypeStruct((NPAD, DHH), jnp.float32),
              jax.ShapeDtypeStruct((NPAD, DHH), jnp.float32),
              jax.ShapeDtypeStruct((SROWS, 128), jnp.float32)),
    mesh=plsc.VectorSubcoreMesh(core_axis_name="c", subcore_axis_name="s",
                                num_cores=NC, num_subcores=NS),
    compiler_params=pltpu.CompilerParams(needs_layout_passes=False),
    scratch_types=[
        pltpu.VMEM((N,), jnp.float32),
        pltpu.VMEM((N,), jnp.float32),
        pltpu.VMEM((SROWS, 128), jnp.float32),
        pltpu.VMEM((3 * B,), jnp.int32),
        pltpu.VMEM((3 * B,), jnp.int32),
        pltpu.VMEM((B,), jnp.float32),
        pltpu.VMEM((B,), jnp.float32),
        pltpu.VMEM((B,), jnp.int32),
        pltpu.VMEM((B,), jnp.int32),
        pltpu.VMEM((B, DHH), jnp.float32),
        pltpu.VMEM((B, DHH), jnp.float32),
        pltpu.VMEM((SROWS,), jnp.int32),
        pltpu.VMEM((2, L), jnp.float32),
        pltpu.VMEM_SHARED((N, DHH), jnp.float32),
        pltpu.VMEM_SHARED((SROWS, 128), jnp.float32),
        pltpu.SemaphoreType.DMA,
        pltpu.SemaphoreType.DMA,
        pltpu.SemaphoreType.DMA,
        pltpu.SemaphoreType.DMA,
        pltpu.SemaphoreType.DMA,
        pltpu.SemaphoreType.DMA,
    ],
)(_sc_body)


def _sc_aggregate(z, edata, asv, adv, c1v, c0v):
    zcat = jnp.concatenate([z[:, :DHH], z[:, DHH:]], axis=0)
    msg0, msg1, s2d = _sc_call(zcat, edata, asv, adv, c1v, c0v)
    s = s2d.reshape(NPAD)[:N].reshape(N, 1)
    return msg0, msg1, s


def kernel(h, e_w, snorm_n, snorm_e, edge_index, Wh, bh, We, be,
           Wself1, Wfunc1, Wattn1, Wself2, Wfunc2, Wattn2, W1, b1):
    edata = jnp.stack([edge_index[0], edge_index[1],
                       lax.bitcast_convert_type(e_w[:, 0], jnp.int32)])
    edata = jnp.pad(edata.reshape(3, NS, EVALID),
                    ((0, 0), (0, 0), (0, EPT - EVALID)))
    edata = (edata.reshape(3, NS, NBLK, B)
             .transpose(1, 2, 0, 3).reshape(NS * NBLK * 3 * B))

    wa1 = jnp.stack([Wattn1[0, :DH], Wattn1[0, DH:2 * DH]], axis=1)
    wa2 = jnp.stack([Wattn2[0, :DH], Wattn2[0, DH:2 * DH]], axis=1)
    c1_1 = Wattn1[0, 2 * DH:] @ We[:, 0]
    c0_1 = Wattn1[0, 2 * DH:] @ be
    c1_2 = Wattn2[0, 2 * DH:] @ We[:, 0]
    c0_2 = Wattn2[0, 2 * DH:] @ be
    c1v1 = jnp.full((L,), c1_1, jnp.float32)
    c0v1 = jnp.full((L,), c0_1, jnp.float32)
    c1v2 = jnp.full((L,), c1_2, jnp.float32)
    c0v2 = jnp.full((L,), c0_2, jnp.float32)

    h0, hs1, z1, aa1 = _stage0(h, Wh, bh, Wself1, Wfunc1, wa1)
    m1a, m1b, s1 = _sc_aggregate(z1, edata, aa1[:, 0], aa1[:, 1], c1v1, c0v1)
    h1, hs2, z2, aa2 = _stage1(h0, hs1, m1a, m1b, s1, Wself2, Wfunc2, wa2)
    m2a, m2b, s2 = _sc_aggregate(z2, edata, aa2[:, 0], aa2[:, 1], c1v2, c0v2)
    y = _stage2(h1, hs2, m2a, m2b, s2, W1, b1)
    return y


# speedup vs baseline: 17.1221x; 1.1026x over previous
import functools

import jax
import jax.numpy as jnp
from jax import lax
from jax.experimental import pallas as pl
from jax.experimental.pallas import tpu as pltpu
from jax.experimental.pallas import tpu_sc as plsc

N = 10000
E = 320000
D_IN = 128
DH = 256
DHH = 128
D_OUT = 128

NC = 2
NS = 16
L = 16

EVALID = E // NS
B = 64
NBLK = 314
EPT = NBLK * B
E_PAD = EPT * NS
NPAD = 10240
WPT = NPAD // NS
SROWS = NPAD // 128

BN = 1000


def _f32(x):
    return jnp.asarray(x, jnp.float32)


def _mm_t(x, w):
    return lax.dot_general(x, w, (((1,), (1,)), ((), ())),
                           preferred_element_type=jnp.float32)


def _stage0_body(h_ref, Wh_ref, bh_ref, Ws_ref, Wf_ref, wa_ref,
                 h0_ref, hs_ref, z_ref, aa_ref):
    h0 = _mm_t(h_ref[...], Wh_ref[...]) + bh_ref[...][None, :]
    h0_ref[...] = h0
    hs_ref[...] = _mm_t(h0, Ws_ref[...])
    z = _mm_t(h0, Wf_ref[...])
    z_ref[0] = z[:, :DHH]
    z_ref[1] = z[:, DHH:]
    aa_ref[...] = jnp.dot(z, wa_ref[...], preferred_element_type=jnp.float32)


def _combine(x, hs, m0, m1, s):
    sgood = s > 0.0
    inv = 1.0 / jnp.where(sgood, s, 1.0)
    msg = jnp.concatenate([m0 * inv, m1 * inv], axis=1)
    upd = jnp.where(sgood, hs + msg, x)
    return x + jnp.maximum(upd, 0.0)


def _stage1_body(x_ref, hs1_ref, m0_ref, m1_ref, s_ref, Ws_ref, Wf_ref, wa_ref,
                 h1_ref, hs_ref, z_ref, aa_ref):
    h1 = _combine(x_ref[...], hs1_ref[...], m0_ref[...], m1_ref[...], s_ref[...])
    h1_ref[...] = h1
    hs_ref[...] = _mm_t(h1, Ws_ref[...])
    z = _mm_t(h1, Wf_ref[...])
    z_ref[0] = z[:, :DHH]
    z_ref[1] = z[:, DHH:]
    aa_ref[...] = jnp.dot(z, wa_ref[...], preferred_element_type=jnp.float32)


def _stage2_body(x_ref, hs2_ref, m0_ref, m1_ref, s_ref, W1_ref, b1_ref, y_ref):
    h2 = _combine(x_ref[...], hs2_ref[...], m0_ref[...], m1_ref[...], s_ref[...])
    y_ref[...] = _mm_t(h2, W1_ref[...]) + b1_ref[...][None, :]


def _row_spec(width):
    return pl.BlockSpec((BN, width), lambda i: (i, 0))


def _w_spec(shape):
    return pl.BlockSpec(shape, lambda i: (0,) * len(shape))


def _stage0(h, Wh, bh, Wself, Wfunc, wa):
    return pl.pallas_call(
        _stage0_body,
        grid=(N // BN,),
        in_specs=[_row_spec(D_IN), _w_spec(Wh.shape), _w_spec(bh.shape),
                  _w_spec(Wself.shape), _w_spec(Wfunc.shape), _w_spec(wa.shape)],
        out_specs=[_row_spec(DH), _row_spec(DH),
                   pl.BlockSpec((2, BN, DHH), lambda i: (0, i, 0)),
                   _row_spec(2)],
        out_shape=[jax.ShapeDtypeStruct((N, DH), jnp.float32),
                   jax.ShapeDtypeStruct((N, DH), jnp.float32),
                   jax.ShapeDtypeStruct((2, N, DHH), jnp.float32),
                   jax.ShapeDtypeStruct((N, 2), jnp.float32)],
    )(h, Wh, bh, Wself, Wfunc, wa)


def _stage1(x, hs1, msg0, msg1, s, Wself, Wfunc, wa):
    return pl.pallas_call(
        _stage1_body,
        grid=(N // BN,),
        in_specs=[_row_spec(DH), _row_spec(DH), _row_spec(DHH), _row_spec(DHH),
                  _row_spec(1),
                  _w_spec(Wself.shape), _w_spec(Wfunc.shape), _w_spec(wa.shape)],
        out_specs=[_row_spec(DH), _row_spec(DH),
                   pl.BlockSpec((2, BN, DHH), lambda i: (0, i, 0)),
                   _row_spec(2)],
        out_shape=[jax.ShapeDtypeStruct((N, DH), jnp.float32),
                   jax.ShapeDtypeStruct((N, DH), jnp.float32),
                   jax.ShapeDtypeStruct((2, N, DHH), jnp.float32),
                   jax.ShapeDtypeStruct((N, 2), jnp.float32)],
    )(x, hs1, msg0, msg1, s, Wself, Wfunc, wa)


def _stage2(x, hs2, msg0, msg1, s, W1, b1):
    return pl.pallas_call(
        _stage2_body,
        grid=(N // BN,),
        in_specs=[_row_spec(DH), _row_spec(DH), _row_spec(DHH), _row_spec(DHH),
                  _row_spec(1),
                  _w_spec(W1.shape), _w_spec(b1.shape)],
        out_specs=_row_spec(D_OUT),
        out_shape=jax.ShapeDtypeStruct((N, D_OUT), jnp.float32),
    )(x, hs2, msg0, msg1, s, W1, b1)


def _sc_body(zcat, edata, asv, adv, c1v_h, c0v_h,
             msg0_out, msg1_out, s_out,
             as_l, ad_l, s_l,
             ed0, ed1, p0, p1, si0, si1, rows0, rows1,
             idn, cvec,
             msg_acc, s_sh,
             esem0, esem1, gsem0, gsem1, ssem0, ssem1):
    c = lax.axis_index("c")
    t = lax.axis_index("s")

    eds = (ed0, ed1)
    pbs = (p0, p1)
    sis = (si0, si1)
    rws = (rows0, rows1)
    esems = (esem0, esem1)
    gsems = (gsem0, gsem1)
    ssems = (ssem0, ssem1)

    zeros16 = jnp.zeros((L,), jnp.float32)

    def _rows_row(i, _):
        for j in range(DHH // L):
            rows0[i, pl.ds(j * L, L)] = zeros16
        return 0
    lax.fori_loop(0, B, _rows_row, 0)

    def _sl_row(i, _):
        for j in range(DHH // L):
            s_l[i, pl.ds(j * L, L)] = zeros16
        return 0
    lax.fori_loop(0, SROWS, _sl_row, 0)

    base_iota = lax.broadcasted_iota(jnp.int32, (L,), 0)
    for g in range(SROWS // L):
        idn[pl.ds(g * L, L)] = base_iota + (g * L)

    @pl.when(t < NS - 1)
    def _():
        for k in range(WPT // B):
            pltpu.sync_copy(rows0, msg_acc.at[pl.ds(t * WPT + k * B, B)])

    @pl.when(t == NS - 1)
    def _():
        tail0 = N - (NS - 1) * WPT
        for k in range(tail0 // B):
            pltpu.sync_copy(rows0, msg_acc.at[pl.ds((NS - 1) * WPT + k * B, B)])
        rem = tail0 % B
        if rem:
            pltpu.sync_copy(rows0.at[pl.ds(0, rem)],
                            msg_acc.at[pl.ds((NS - 1) * WPT + (tail0 // B) * B,
                                             rem)])

    @pl.when(t < 10)
    def _():
        pltpu.sync_copy(rows0.at[pl.ds(0, 8)], s_sh.at[pl.ds(t * 8, 8)])

    pltpu.sync_copy(asv, as_l)
    pltpu.sync_copy(adv, ad_l)
    pltpu.sync_copy(c1v_h, cvec.at[0])
    pltpu.sync_copy(c0v_h, cvec.at[1])
    c1vec = cvec[0, :]
    c0vec = cvec[1, :]

    plsc.subcore_barrier()

    eoff0 = t * EPT
    zbase = c * N

    def issue_edges(b, k):
        pltpu.async_copy(edata.at[pl.ds((t * NBLK + b) * 3 * B, 3 * B)],
                         eds[k], esems[k])

    def wait_edges(b, k):
        pltpu.make_async_copy(edata.at[pl.ds((t * NBLK + b) * 3 * B, 3 * B)],
                              eds[k], esems[k]).wait()

    def scalar_phase(b, k):
        ed = eds[k]
        for g in range(B // L):
            sl = pl.ds(g * L, L)
            srcv = ed[pl.ds(g * L, L)]
            dstv = ed[pl.ds(B + g * L, L)]
            eww = plsc.bitcast(ed[pl.ds(2 * B + g * L, L)], jnp.int32)
            a_s = plsc.load_gather(as_l, [srcv])
            a_d = plsc.load_gather(ad_l, [dstv])
            sc_ = a_s + a_d + eww * c1vec + c0vec
            lr = jnp.where(sc_ >= 0.0, sc_, sc_ * 0.01)
            p = jnp.exp(lr)
            lid = b * B + g * L + base_iota
            p = jnp.where(lid < EVALID, p, 0.0)
            pbs[k][sl] = p
            plsc.addupdate_scatter(s_l, [dstv >> 7, dstv & 127], p)
            sis[k][sl] = dstv
            ed[pl.ds(g * L, L)] = srcv + zbase
        pltpu.async_copy(zcat.at[ed.at[pl.ds(0, B)]], rws[k], gsems[k])

    def wait_gather(k):
        pltpu.make_async_copy(zcat.at[eds[k].at[pl.ds(0, B)]], rws[k],
                              gsems[k]).wait()

    def scale_and_scatter(k):
        rw = rws[k]
        for g in range(B // L):
            pv = pbs[k][pl.ds(g * L, L)]
            for i in range(L):
                spl = jnp.broadcast_to(pv[i], (L,))
                r = g * L + i
                for j in range(DHH // L):
                    rw[r, pl.ds(j * L, L)] = rw[r, pl.ds(j * L, L)] * spl
        pltpu.async_copy(rw, msg_acc.at[sis[k]], ssems[k], add=True)

    def wait_scatter(k):
        pltpu.make_async_copy(rws[k], msg_acc.at[sis[k]], ssems[k]).wait()

    def stage(j, A, Bn):
        @pl.when(j >= 1)
        def _():
            wait_scatter(Bn)

        @pl.when(j + 1 < NBLK)
        def _():
            wait_edges(j + 1, Bn)
            scalar_phase(j + 1, Bn)

        wait_gather(A)
        scale_and_scatter(A)

        @pl.when(j + 2 < NBLK)
        def _():
            issue_edges(j + 2, A)

    issue_edges(0, 0)
    issue_edges(1, 1)
    wait_edges(0, 0)
    scalar_phase(0, 0)

    def _pair(i, _):
        stage(2 * i, 0, 1)
        stage(2 * i + 1, 1, 0)
        return 0
    lax.fori_loop(0, NBLK // 2, _pair, 0)

    wait_scatter(1)

    plsc.subcore_barrier()
    pltpu.sync_copy(s_l, s_sh.at[idn], add=True)
    plsc.subcore_barrier()

    tail = N - (NS - 1) * WPT
    for half, mref in ((0, msg0_out), (1, msg1_out)):
        @pl.when((c == half) & (t < NS - 1))
        def _(mref=mref):
            pltpu.sync_copy(msg_acc.at[pl.ds(t * WPT, WPT)],
                            mref.at[pl.ds(t * WPT, WPT)])

        @pl.when((c == half) & (t == NS - 1))
        def _(mref=mref):
            pltpu.sync_copy(msg_acc.at[pl.ds((NS - 1) * WPT, tail)],
                            mref.at[pl.ds((NS - 1) * WPT, tail)])

    @pl.when((c == 0) & (t < 10))
    def _():
        pltpu.sync_copy(s_sh.at[pl.ds(t * 8, 8)], s_out.at[pl.ds(t * 8, 8)])


_sc_call = functools.partial(
    pl.kernel,
    out_type=(jax.ShapeDtypeStruct((NPAD, DHH), jnp.float32),
              jax.ShapeDtypeStruct((NPAD, DHH), jnp.float32),
              jax.ShapeDtypeStruct((SROWS, 128), jnp.float32)),
    mesh=plsc.VectorSubcoreMesh(core_axis_name="c", subcore_axis_name="s",
                                num_cores=NC, num_subcores=NS),
    compiler_params=pltpu.CompilerParams(needs_layout_passes=False),
    scratch_types=[
        pltpu.VMEM((N,), jnp.float32),
        pltpu.VMEM((N,), jnp.float32),
        pltpu.VMEM((SROWS, 128), jnp.float32),
        pltpu.VMEM((3 * B,), jnp.int32),
        pltpu.VMEM((3 * B,), jnp.int32),
        pltpu.VMEM((B,), jnp.float32),
        pltpu.VMEM((B,), jnp.float32),
        pltpu.VMEM((B,), jnp.int32),
        pltpu.VMEM((B,), jnp.int32),
        pltpu.VMEM((B, DHH), jnp.float32),
        pltpu.VMEM((B, DHH), jnp.float32),
        pltpu.VMEM((SROWS,), jnp.int32),
        pltpu.VMEM((2, L), jnp.float32),
        pltpu.VMEM_SHARED((N, DHH), jnp.float32),
        pltpu.VMEM_SHARED((SROWS, 128), jnp.float32),
        pltpu.SemaphoreType.DMA,
        pltpu.SemaphoreType.DMA,
        pltpu.SemaphoreType.DMA,
        pltpu.SemaphoreType.DMA,
        pltpu.SemaphoreType.DMA,
        pltpu.SemaphoreType.DMA,
    ],
)(_sc_body)


def _sc_aggregate(z, edata, asv, adv, c1v, c0v):
    zcat = z.reshape(2 * N, DHH)
    msg0, msg1, s2d = _sc_call(zcat, edata, asv, adv, c1v, c0v)
    s = s2d.reshape(NPAD)[:N].reshape(N, 1)
    return msg0, msg1, s


def kernel(h, e_w, snorm_n, snorm_e, edge_index, Wh, bh, We, be,
           Wself1, Wfunc1, Wattn1, Wself2, Wfunc2, Wattn2, W1, b1):
    edata = jnp.stack([edge_index[0], edge_index[1],
                       lax.bitcast_convert_type(e_w[:, 0], jnp.int32)])
    edata = jnp.pad(edata.reshape(3, NS, EVALID),
                    ((0, 0), (0, 0), (0, EPT - EVALID)))
    edata = (edata.reshape(3, NS, NBLK, B)
             .transpose(1, 2, 0, 3).reshape(NS * NBLK * 3 * B))

    wa1 = jnp.stack([Wattn1[0, :DH], Wattn1[0, DH:2 * DH]], axis=1)
    wa2 = jnp.stack([Wattn2[0, :DH], Wattn2[0, DH:2 * DH]], axis=1)
    c1_1 = Wattn1[0, 2 * DH:] @ We[:, 0]
    c0_1 = Wattn1[0, 2 * DH:] @ be
    c1_2 = Wattn2[0, 2 * DH:] @ We[:, 0]
    c0_2 = Wattn2[0, 2 * DH:] @ be
    c1v1 = jnp.full((L,), c1_1, jnp.float32)
    c0v1 = jnp.full((L,), c0_1, jnp.float32)
    c1v2 = jnp.full((L,), c1_2, jnp.float32)
    c0v2 = jnp.full((L,), c0_2, jnp.float32)

    h0, hs1, z1, aa1 = _stage0(h, Wh, bh, Wself1, Wfunc1, wa1)
    m1a, m1b, s1 = _sc_aggregate(z1, edata, aa1[:, 0], aa1[:, 1], c1v1, c0v1)
    h1, hs2, z2, aa2 = _stage1(h0, hs1, m1a, m1b, s1, Wself2, Wfunc2, wa2)
    m2a, m2b, s2 = _sc_aggregate(z2, edata, aa2[:, 0], aa2[:, 1], c1v2, c0v2)
    y = _stage2(h1, hs2, m2a, m2b, s2, W1, b1)
    return y


# speedup vs baseline: 17.2880x; 1.0097x over previous
import functools

import jax
import jax.numpy as jnp
from jax import lax
from jax.experimental import pallas as pl
from jax.experimental.pallas import tpu as pltpu
from jax.experimental.pallas import tpu_sc as plsc

N = 10000
E = 320000
D_IN = 128
DH = 256
DHH = 128
D_OUT = 128

NC = 2
NS = 16
L = 16

EVALID = E // NS
B = 64
NBLK = 314
EPT = NBLK * B
E_PAD = EPT * NS
NPAD = 10240
WPT = NPAD // NS
SROWS = NPAD // 128

BN = 1000


def _f32(x):
    return jnp.asarray(x, jnp.float32)


def _mm_t(x, w):
    return lax.dot_general(x, w, (((1,), (1,)), ((), ())),
                           preferred_element_type=jnp.float32)


def _stage0_body(h_ref, Wh_ref, bh_ref, Ws_ref, Wf_ref, wa_ref,
                 h0_ref, hs_ref, z_ref, aa_ref):
    h0 = _mm_t(h_ref[...], Wh_ref[...]) + bh_ref[...][None, :]
    h0_ref[...] = h0
    hs_ref[...] = _mm_t(h0, Ws_ref[...])
    z = _mm_t(h0, Wf_ref[...])
    z_ref[0] = z[:, :DHH]
    z_ref[1] = z[:, DHH:]
    aa_ref[...] = jnp.dot(z, wa_ref[...], preferred_element_type=jnp.float32)


def _combine(x, hs, m0, m1, s):
    sgood = s > 0.0
    inv = 1.0 / jnp.where(sgood, s, 1.0)
    msg = jnp.concatenate([m0 * inv, m1 * inv], axis=1)
    upd = jnp.where(sgood, hs + msg, x)
    return x + jnp.maximum(upd, 0.0)


def _stage1_body(x_ref, hs1_ref, m0_ref, m1_ref, s_ref, Ws_ref, Wf_ref, wa_ref,
                 h1_ref, hs_ref, z_ref, aa_ref):
    h1 = _combine(x_ref[...], hs1_ref[...], m0_ref[...], m1_ref[...], s_ref[...])
    h1_ref[...] = h1
    hs_ref[...] = _mm_t(h1, Ws_ref[...])
    z = _mm_t(h1, Wf_ref[...])
    z_ref[0] = z[:, :DHH]
    z_ref[1] = z[:, DHH:]
    aa_ref[...] = jnp.dot(z, wa_ref[...], preferred_element_type=jnp.float32)


def _stage2_body(x_ref, hs2_ref, m0_ref, m1_ref, s_ref, W1_ref, b1_ref, y_ref):
    h2 = _combine(x_ref[...], hs2_ref[...], m0_ref[...], m1_ref[...], s_ref[...])
    y_ref[...] = _mm_t(h2, W1_ref[...]) + b1_ref[...][None, :]


def _row_spec(width):
    return pl.BlockSpec((BN, width), lambda i: (i, 0))


def _w_spec(shape):
    return pl.BlockSpec(shape, lambda i: (0,) * len(shape))


def _stage0(h, Wh, bh, Wself, Wfunc, wa):
    return pl.pallas_call(
        _stage0_body,
        grid=(N // BN,),
        in_specs=[_row_spec(D_IN), _w_spec(Wh.shape), _w_spec(bh.shape),
                  _w_spec(Wself.shape), _w_spec(Wfunc.shape), _w_spec(wa.shape)],
        out_specs=[_row_spec(DH), _row_spec(DH),
                   pl.BlockSpec((2, BN, DHH), lambda i: (0, i, 0)),
                   _row_spec(2)],
        out_shape=[jax.ShapeDtypeStruct((N, DH), jnp.float32),
                   jax.ShapeDtypeStruct((N, DH), jnp.float32),
                   jax.ShapeDtypeStruct((2, N, DHH), jnp.float32),
                   jax.ShapeDtypeStruct((N, 2), jnp.float32)],
    )(h, Wh, bh, Wself, Wfunc, wa)


def _stage1(x, hs1, msg0, msg1, s, Wself, Wfunc, wa):
    return pl.pallas_call(
        _stage1_body,
        grid=(N // BN,),
        in_specs=[_row_spec(DH), _row_spec(DH), _row_spec(DHH), _row_spec(DHH),
                  _row_spec(1),
                  _w_spec(Wself.shape), _w_spec(Wfunc.shape), _w_spec(wa.shape)],
        out_specs=[_row_spec(DH), _row_spec(DH),
                   pl.BlockSpec((2, BN, DHH), lambda i: (0, i, 0)),
                   _row_spec(2)],
        out_shape=[jax.ShapeDtypeStruct((N, DH), jnp.float32),
                   jax.ShapeDtypeStruct((N, DH), jnp.float32),
                   jax.ShapeDtypeStruct((2, N, DHH), jnp.float32),
                   jax.ShapeDtypeStruct((N, 2), jnp.float32)],
    )(x, hs1, msg0, msg1, s, Wself, Wfunc, wa)


def _stage2(x, hs2, msg0, msg1, s, W1, b1):
    return pl.pallas_call(
        _stage2_body,
        grid=(N // BN,),
        in_specs=[_row_spec(DH), _row_spec(DH), _row_spec(DHH), _row_spec(DHH),
                  _row_spec(1),
                  _w_spec(W1.shape), _w_spec(b1.shape)],
        out_specs=_row_spec(D_OUT),
        out_shape=jax.ShapeDtypeStruct((N, D_OUT), jnp.float32),
    )(x, hs2, msg0, msg1, s, W1, b1)


def _sc_body(zcat, edata, aav, c1v_h, c0v_h,
             msg0_out, msg1_out, s_out,
             aa_l, s_l,
             ed0, ed1, p0, p1, si0, si1, rows0, rows1,
             idn, cvec,
             msg_acc, s_sh,
             esem0, esem1, gsem0, gsem1, hsem0, hsem1, ssem0, ssem1):
    c = lax.axis_index("c")
    t = lax.axis_index("s")

    eds = (ed0, ed1)
    pbs = (p0, p1)
    sis = (si0, si1)
    rws = (rows0, rows1)
    esems = (esem0, esem1)
    gsems = (gsem0, gsem1)
    hsems = (hsem0, hsem1)
    ssems = (ssem0, ssem1)

    zeros16 = jnp.zeros((L,), jnp.float32)

    def _rows_row(i, _):
        for j in range(DHH // L):
            rows0[i, pl.ds(j * L, L)] = zeros16
        return 0
    lax.fori_loop(0, B, _rows_row, 0)

    def _sl_row(i, _):
        for j in range(DHH // L):
            s_l[i, pl.ds(j * L, L)] = zeros16
        return 0
    lax.fori_loop(0, SROWS, _sl_row, 0)

    base_iota = lax.broadcasted_iota(jnp.int32, (L,), 0)
    for g in range(SROWS // L):
        idn[pl.ds(g * L, L)] = base_iota + (g * L)

    @pl.when(t < NS - 1)
    def _():
        for k in range(WPT // B):
            pltpu.sync_copy(rows0, msg_acc.at[pl.ds(t * WPT + k * B, B)])

    @pl.when(t == NS - 1)
    def _():
        tail0 = N - (NS - 1) * WPT
        for k in range(tail0 // B):
            pltpu.sync_copy(rows0, msg_acc.at[pl.ds((NS - 1) * WPT + k * B, B)])
        rem = tail0 % B
        if rem:
            pltpu.sync_copy(rows0.at[pl.ds(0, rem)],
                            msg_acc.at[pl.ds((NS - 1) * WPT + (tail0 // B) * B,
                                             rem)])

    @pl.when(t < 10)
    def _():
        pltpu.sync_copy(rows0.at[pl.ds(0, 8)], s_sh.at[pl.ds(t * 8, 8)])

    pltpu.sync_copy(aav, aa_l)
    pltpu.sync_copy(c1v_h, cvec.at[0])
    pltpu.sync_copy(c0v_h, cvec.at[1])
    c1vec = cvec[0, :]
    c0vec = cvec[1, :]

    plsc.subcore_barrier()

    eoff0 = t * EPT
    zbase = c * N

    def issue_edges(b, k):
        pltpu.async_copy(edata.at[pl.ds((t * NBLK + b) * 3 * B, 3 * B)],
                         eds[k], esems[k])

    def wait_edges(b, k):
        pltpu.make_async_copy(edata.at[pl.ds((t * NBLK + b) * 3 * B, 3 * B)],
                              eds[k], esems[k]).wait()

    def scalar_phase(b, k):
        ed = eds[k]
        for g in range(B // L):
            sl = pl.ds(g * L, L)
            srcv = ed[pl.ds(g * L, L)]
            dstv = ed[pl.ds(B + g * L, L)]
            eww = plsc.bitcast(ed[pl.ds(2 * B + g * L, L)], jnp.int32)
            a_s = plsc.load_gather(aa_l, [srcv << 1])
            a_d = plsc.load_gather(aa_l, [(dstv << 1) | 1])
            sc_ = a_s + a_d + eww * c1vec + c0vec
            lr = jnp.where(sc_ >= 0.0, sc_, sc_ * 0.01)
            p = jnp.exp(lr)
            lid = b * B + g * L + base_iota
            p = jnp.where(lid < EVALID, p, 0.0)
            pbs[k][sl] = p
            plsc.addupdate_scatter(s_l, [dstv >> 7, dstv & 127], p)
            sis[k][sl] = dstv
            ed[pl.ds(g * L, L)] = srcv + zbase
        H = B // 2
        pltpu.async_copy(zcat.at[ed.at[pl.ds(0, H)]],
                         rws[k].at[pl.ds(0, H)], gsems[k])
        pltpu.async_copy(zcat.at[ed.at[pl.ds(H, H)]],
                         rws[k].at[pl.ds(H, H)], hsems[k])

    def wait_gather_half(k, h):
        H = B // 2
        sem = gsems[k] if h == 0 else hsems[k]
        pltpu.make_async_copy(zcat.at[eds[k].at[pl.ds(h * H, H)]],
                              rws[k].at[pl.ds(h * H, H)], sem).wait()

    def scale_and_scatter(k):
        rw = rws[k]
        for h in range(2):
            wait_gather_half(k, h)
            for g in range(h * B // (2 * L), (h + 1) * B // (2 * L)):
                pv = pbs[k][pl.ds(g * L, L)]
                for i in range(L):
                    spl = jnp.broadcast_to(pv[i], (L,))
                    r = g * L + i
                    for j in range(DHH // L):
                        rw[r, pl.ds(j * L, L)] = rw[r, pl.ds(j * L, L)] * spl
        pltpu.async_copy(rw, msg_acc.at[sis[k]], ssems[k], add=True)

    def wait_scatter(k):
        pltpu.make_async_copy(rws[k], msg_acc.at[sis[k]], ssems[k]).wait()

    def stage(j, A, Bn):
        @pl.when(j >= 1)
        def _():
            wait_scatter(Bn)

        @pl.when(j + 1 < NBLK)
        def _():
            wait_edges(j + 1, Bn)
            scalar_phase(j + 1, Bn)

        scale_and_scatter(A)

        @pl.when(j + 2 < NBLK)
        def _():
            issue_edges(j + 2, A)

    issue_edges(0, 0)
    issue_edges(1, 1)
    wait_edges(0, 0)
    scalar_phase(0, 0)

    def _pair(i, _):
        stage(2 * i, 0, 1)
        stage(2 * i + 1, 1, 0)
        return 0
    lax.fori_loop(0, NBLK // 2, _pair, 0)

    wait_scatter(1)

    plsc.subcore_barrier()
    pltpu.sync_copy(s_l, s_sh.at[idn], add=True)
    plsc.subcore_barrier()

    tail = N - (NS - 1) * WPT
    for half, mref in ((0, msg0_out), (1, msg1_out)):
        @pl.when((c == half) & (t < NS - 1))
        def _(mref=mref):
            pltpu.sync_copy(msg_acc.at[pl.ds(t * WPT, WPT)],
                            mref.at[pl.ds(t * WPT, WPT)])

        @pl.when((c == half) & (t == NS - 1))
        def _(mref=mref):
            pltpu.sync_copy(msg_acc.at[pl.ds((NS - 1) * WPT, tail)],
                            mref.at[pl.ds((NS - 1) * WPT, tail)])

    @pl.when((c == 0) & (t < 10))
    def _():
        pltpu.sync_copy(s_sh.at[pl.ds(t * 8, 8)], s_out.at[pl.ds(t * 8, 8)])


_sc_call = functools.partial(
    pl.kernel,
    out_type=(jax.ShapeDtypeStruct((NPAD, DHH), jnp.float32),
              jax.ShapeDtypeStruct((NPAD, DHH), jnp.float32),
              jax.ShapeDtypeStruct((SROWS, 128), jnp.float32)),
    mesh=plsc.VectorSubcoreMesh(core_axis_name="c", subcore_axis_name="s",
                                num_cores=NC, num_subcores=NS),
    compiler_params=pltpu.CompilerParams(needs_layout_passes=False),
    scratch_types=[
        pltpu.VMEM((2 * N,), jnp.float32),
        pltpu.VMEM((SROWS, 128), jnp.float32),
        pltpu.VMEM((3 * B,), jnp.int32),
        pltpu.VMEM((3 * B,), jnp.int32),
        pltpu.VMEM((B,), jnp.float32),
        pltpu.VMEM((B,), jnp.float32),
        pltpu.VMEM((B,), jnp.int32),
        pltpu.VMEM((B,), jnp.int32),
        pltpu.VMEM((B, DHH), jnp.float32),
        pltpu.VMEM((B, DHH), jnp.float32),
        pltpu.VMEM((SROWS,), jnp.int32),
        pltpu.VMEM((2, L), jnp.float32),
        pltpu.VMEM_SHARED((N, DHH), jnp.float32),
        pltpu.VMEM_SHARED((SROWS, 128), jnp.float32),
        pltpu.SemaphoreType.DMA,
        pltpu.SemaphoreType.DMA,
        pltpu.SemaphoreType.DMA,
        pltpu.SemaphoreType.DMA,
        pltpu.SemaphoreType.DMA,
        pltpu.SemaphoreType.DMA,
        pltpu.SemaphoreType.DMA,
        pltpu.SemaphoreType.DMA,
    ],
)(_sc_body)


def _sc_aggregate(z, edata, aa, c1v, c0v):
    zcat = z.reshape(2 * N, DHH)
    msg0, msg1, s2d = _sc_call(zcat, edata, aa.reshape(2 * N), c1v, c0v)
    s = s2d.reshape(NPAD)[:N].reshape(N, 1)
    return msg0, msg1, s


def kernel(h, e_w, snorm_n, snorm_e, edge_index, Wh, bh, We, be,
           Wself1, Wfunc1, Wattn1, Wself2, Wfunc2, Wattn2, W1, b1):
    edata = jnp.stack([edge_index[0], edge_index[1],
                       lax.bitcast_convert_type(e_w[:, 0], jnp.int32)])
    edata = jnp.pad(edata.reshape(3, NS, EVALID),
                    ((0, 0), (0, 0), (0, EPT - EVALID)))
    edata = (edata.reshape(3, NS, NBLK, B)
             .transpose(1, 2, 0, 3).reshape(NS * NBLK * 3 * B))

    wa1 = jnp.stack([Wattn1[0, :DH], Wattn1[0, DH:2 * DH]], axis=1)
    wa2 = jnp.stack([Wattn2[0, :DH], Wattn2[0, DH:2 * DH]], axis=1)
    c1_1 = Wattn1[0, 2 * DH:] @ We[:, 0]
    c0_1 = Wattn1[0, 2 * DH:] @ be
    c1_2 = Wattn2[0, 2 * DH:] @ We[:, 0]
    c0_2 = Wattn2[0, 2 * DH:] @ be
    c1v1 = jnp.full((L,), c1_1, jnp.float32)
    c0v1 = jnp.full((L,), c0_1, jnp.float32)
    c1v2 = jnp.full((L,), c1_2, jnp.float32)
    c0v2 = jnp.full((L,), c0_2, jnp.float32)

    h0, hs1, z1, aa1 = _stage0(h, Wh, bh, Wself1, Wfunc1, wa1)
    m1a, m1b, s1 = _sc_aggregate(z1, edata, aa1, c1v1, c0v1)
    h1, hs2, z2, aa2 = _stage1(h0, hs1, m1a, m1b, s1, Wself2, Wfunc2, wa2)
    m2a, m2b, s2 = _sc_aggregate(z2, edata, aa2, c1v2, c0v2)
    y = _stage2(h1, hs2, m2a, m2b, s2, W1, b1)
    return y
